# baseline (speedup 1.0000x reference)
import functools

import jax
import jax.numpy as jnp
from jax import lax
from jax.experimental import pallas as pl
from jax.experimental.pallas import tpu as pltpu
from jax.experimental.pallas import tpu_sc as plsc

EMBED = 16
BATCH = 16384
NC = 2
NS = 16
NW = NC * NS
BPW = BATCH // NW
GRP = 128
NG = BPW // GRP

_mesh = plsc.VectorSubcoreMesh(core_axis_name="c", subcore_axis_name="s")


@functools.partial(
    pl.kernel,
    mesh=_mesh,
    compiler_params=pltpu.CompilerParams(use_tc_tiling_on_sc=False),
    out_type=[
        jax.ShapeDtypeStruct((NW, EMBED), jnp.float32),
        jax.ShapeDtypeStruct((BATCH,), jnp.float32),
    ],
    scratch_types=[
        pltpu.VMEM((NG, GRP), jnp.int32),
        pltpu.VMEM((NG, GRP), jnp.int32),
        pltpu.VMEM((BPW, EMBED), jnp.float32),
        pltpu.VMEM((BPW, EMBED), jnp.float32),
        pltpu.VMEM((BPW,), jnp.float32),
        pltpu.VMEM((BPW,), jnp.float32),
        pltpu.VMEM((EMBED,), jnp.float32),
        pltpu.VMEM((BPW,), jnp.float32),
        pltpu.SemaphoreType.DMA,
    ],
)
def _sc_gather_dot(
    uidx_hbm, midx_hbm, uemb_hbm, ubias_hbm, memb_hbm, mbias_hbm,
    part_out, bsum_out,
    uidx_v, midx_v, urows_v, mrows_v, ub_v, mb_v, acc_v, bs_v, sem,
):
    wid = lax.axis_index("s") * NC + lax.axis_index("c")
    base = wid * BPW

    pltpu.sync_copy(uidx_hbm.at[wid], uidx_v)
    pltpu.sync_copy(midx_hbm.at[wid], midx_v)

    for j in range(NG):
        rows = pl.ds(j * GRP, GRP)
        pltpu.async_copy(uemb_hbm.at[uidx_v.at[j]], urows_v.at[rows], sem)
        pltpu.async_copy(memb_hbm.at[midx_v.at[j]], mrows_v.at[rows], sem)
        pltpu.async_copy(ubias_hbm.at[uidx_v.at[j]], ub_v.at[rows], sem)
        pltpu.async_copy(mbias_hbm.at[midx_v.at[j]], mb_v.at[rows], sem)
    for j in range(NG):
        rows = pl.ds(j * GRP, GRP)
        pltpu.make_async_copy(uemb_hbm.at[uidx_v.at[j]], urows_v.at[rows], sem).wait()
        pltpu.make_async_copy(memb_hbm.at[midx_v.at[j]], mrows_v.at[rows], sem).wait()
        pltpu.make_async_copy(ubias_hbm.at[uidx_v.at[j]], ub_v.at[rows], sem).wait()
        pltpu.make_async_copy(mbias_hbm.at[midx_v.at[j]], mb_v.at[rows], sem).wait()

    def body(i, acc):
        return acc + urows_v[i, :] * mrows_v[i, :]

    acc_v[...] = lax.fori_loop(0, BPW, body, jnp.zeros((EMBED,), jnp.float32))
    pltpu.sync_copy(acc_v, part_out.at[wid])

    for j in range(BPW // 16):
        sl = pl.ds(j * 16, 16)
        bs_v[sl] = ub_v[sl] + mb_v[sl]
    pltpu.sync_copy(bs_v, bsum_out.at[pl.ds(base, BPW)])


def _tc_finish(part_ref, bias_ref, out_ref):
    s = jnp.sum(part_ref[...])
    out_ref[...] = jax.nn.sigmoid(bias_ref[...] + s)


def kernel(inputs, user_embedding, user_bias, movie_embedding, movie_bias):
    idx = inputs.astype(jnp.int32)
    uidx = idx[:, 0].reshape(NW, NG, GRP)
    midx = idx[:, 1].reshape(NW, NG, GRP)
    ubias = user_bias.reshape(-1)
    mbias = movie_bias.reshape(-1)

    partials, bsum = _sc_gather_dot(
        uidx, midx, user_embedding, ubias, movie_embedding, mbias
    )

    out = pl.pallas_call(
        _tc_finish,
        out_shape=jax.ShapeDtypeStruct((128, 128), jnp.float32),
    )(partials, bsum.reshape(128, 128))
    return out.reshape(BATCH, 1)


# speedup vs baseline: 1.0053x; 1.0053x over previous
import functools

import jax
import jax.numpy as jnp
from jax import lax
from jax.experimental import pallas as pl
from jax.experimental.pallas import tpu as pltpu
from jax.experimental.pallas import tpu_sc as plsc

EMBED = 16
BATCH = 16384
NC = 2
NS = 16
NW = NC * NS
BPW = BATCH // NW
GRP = 128
NG = BPW // GRP

_mesh = plsc.VectorSubcoreMesh(core_axis_name="c", subcore_axis_name="s")


@functools.partial(
    pl.kernel,
    mesh=_mesh,
    compiler_params=pltpu.CompilerParams(use_tc_tiling_on_sc=False),
    out_type=[
        jax.ShapeDtypeStruct((NW * EMBED,), jnp.float32),
        jax.ShapeDtypeStruct((BATCH,), jnp.float32),
    ],
    scratch_types=[
        pltpu.VMEM((BPW,), jnp.int32),
        pltpu.VMEM((BPW,), jnp.int32),
        pltpu.VMEM((BPW, EMBED), jnp.float32),
        pltpu.VMEM((BPW, EMBED), jnp.float32),
        pltpu.VMEM((EMBED,), jnp.float32),
        pltpu.VMEM((BPW,), jnp.float32),
        pltpu.SemaphoreType.DMA,
    ],
)
def _sc_gather_dot(
    uidx_hbm, midx_hbm, uemb_hbm, memb_hbm,
    part_out, bsum_out,
    uidx_v, midx_v, urows_v, mrows_v, acc_v, bs_v, sem,
):
    wid = lax.axis_index("s") * NC + lax.axis_index("c")
    base = wid * BPW

    pltpu.sync_copy(uidx_hbm.at[pl.ds(base, BPW)], uidx_v)
    pltpu.sync_copy(midx_hbm.at[pl.ds(base, BPW)], midx_v)

    for j in range(NG):
        rows = pl.ds(j * GRP, GRP)
        pltpu.async_copy(uemb_hbm.at[uidx_v.at[rows]], urows_v.at[rows], sem)
        pltpu.async_copy(memb_hbm.at[midx_v.at[rows]], mrows_v.at[rows], sem)
    for j in range(NG):
        rows = pl.ds(j * GRP, GRP)
        pltpu.make_async_copy(uemb_hbm.at[uidx_v.at[rows]], urows_v.at[rows], sem).wait()
        pltpu.make_async_copy(memb_hbm.at[midx_v.at[rows]], mrows_v.at[rows], sem).wait()

    def body(i, acc):
        return acc + urows_v[i, :] * mrows_v[i, :]

    acc_v[...] = lax.fori_loop(0, BPW, body, jnp.zeros((EMBED,), jnp.float32))
    pltpu.sync_copy(acc_v, part_out.at[pl.ds(wid * EMBED, EMBED)])

    for j in range(BPW // 16):
        sl = pl.ds(j * 16, 16)
        bs_v[sl] = jnp.zeros((16,), jnp.float32)
    pltpu.sync_copy(bs_v, bsum_out.at[pl.ds(base, BPW)])


def _tc_finish(part_ref, bias_ref, out_ref):
    s = jnp.sum(part_ref[...])
    out_ref[...] = jax.nn.sigmoid(bias_ref[...] + s)


def kernel(inputs, user_embedding, user_bias, movie_embedding, movie_bias):
    idx = inputs.astype(jnp.int32)
    uidx = idx[:, 0]
    midx = idx[:, 1]

    partials, bsum = _sc_gather_dot(uidx, midx, user_embedding, movie_embedding)

    out = pl.pallas_call(
        _tc_finish,
        out_shape=jax.ShapeDtypeStruct((128, 128), jnp.float32),
    )(partials.reshape(NW, EMBED), bsum.reshape(128, 128))
    return out.reshape(BATCH, 1)
